# Initial kernel scaffold; baseline (speedup 1.0000x reference)
import jax
import jax.numpy as jnp
from jax.experimental import pallas as pl


def kernel(input, dw_w, dw_b, ln_w, ln_b, off_w, off_b, mask_w, mask_b, in_w, in_b, out_w, out_b):
    raise NotImplementedError("write your pallas kernel here")


# speedup vs baseline: 189.0756x; 189.0756x over previous
import functools
import math

import jax
import jax.numpy as jnp
from jax import lax
from jax.experimental import pallas as pl
from jax.experimental.pallas import tpu as pltpu
from jax.experimental.pallas import tpu_sc as plsc

C = 64
G = 4
GC = 16
P = 16
H = 112
HO = 224
NPIX = HO * HO
NROW = H * H

def _proj_body(inp_ref, wt_ref, out_ref):
    out_ref[:] = jnp.dot(inp_ref[:], wt_ref[:], preferred_element_type=jnp.float32)


def _proj_call(inp_flat, in_wt):
    return pl.pallas_call(
        _proj_body,
        out_shape=jax.ShapeDtypeStruct((NROW, C), jnp.float32),
    )(inp_flat, in_wt)


TR = 28
NRC = H // TR


def _fields_body(inpad_ref, wcoef_ref, dwb_ref, lnw_ref, lnb_ref,
                 wox_ref, woy_ref, wm_ref, box_ref, boy_ref, bm_ref, bd_ref,
                 ox_ref, oy_ref, m_ref):
    pb = pl.program_id(0)
    rc = pl.program_id(1)
    pr = pb // 2
    pc = pb % 2
    r0 = rc * TR
    x = jnp.zeros((TR, H, C), jnp.float32)
    for ta in range(2):
        for tb in range(2):
            w = wcoef_ref[pb, ta * 2 + tb, :]
            rs = r0 + 1 + pr - ta
            cs = 1 + pc - tb
            sl = inpad_ref[pl.ds(rs, TR), pl.ds(cs, H), :]
            x = x + sl * w[None, None, :]
    x = x.reshape(TR * H, C) + dwb_ref[:]
    mu = jnp.mean(x, -1, keepdims=True)
    var = jnp.mean((x - mu) * (x - mu), -1, keepdims=True)
    x = (x - mu) * lax.rsqrt(var + 1e-6) * lnw_ref[:] + lnb_ref[:]
    x = 0.5 * x * (1.0 + lax.erf(x * (1.0 / math.sqrt(2.0))))
    ox_ref[:] = jnp.dot(x, wox_ref[:], preferred_element_type=jnp.float32) + box_ref[:]
    oy_ref[:] = jnp.dot(x, woy_ref[:], preferred_element_type=jnp.float32) + boy_ref[:]
    ml = jnp.dot(x, wm_ref[:], preferred_element_type=jnp.float32) + bm_ref[:]
    ml = ml - jnp.max(ml, -1, keepdims=True)
    e = jnp.exp(ml)
    ssum = jnp.dot(e, bd_ref[:], preferred_element_type=jnp.float32)
    m_ref[:] = e / ssum


def _fields_call(inpad, wcoef, dw_b, ln_w, ln_b, wox, woy, wm, box, boy, bm, bd):
    blk = TR * H
    full = lambda shp: pl.BlockSpec(shp, lambda pb, rc: (0,) * len(shp))
    out_spec = pl.BlockSpec((blk, C), lambda pb, rc: (pb * NRC + rc, 0))
    out = jax.ShapeDtypeStruct((NPIX, C), jnp.float32)
    return pl.pallas_call(
        _fields_body,
        grid=(4, NRC),
        in_specs=[
            full((H + 2, H + 2, C)),
            full((4, 4, C)),
            full((C,)), full((C,)), full((C,)),
            full((C, C)), full((C, C)), full((C, C)),
            full((C,)), full((C,)), full((C,)),
            full((C, C)),
        ],
        out_specs=[out_spec, out_spec, out_spec],
        out_shape=[out, out, out],
    )(inpad, wcoef, dw_b, ln_w, ln_b, wox, woy, wm, box, boy, bm, bd)


def _out_body(inp_ref, wt_ref, b_ref, out_ref):
    out_ref[:] = jnp.dot(inp_ref[:], wt_ref[:], preferred_element_type=jnp.float32) + b_ref[:]


def _out_call(dcn, out_wt, out_b):
    blk = NPIX // 16
    return pl.pallas_call(
        _out_body,
        grid=(16,),
        in_specs=[
            pl.BlockSpec((blk, C), lambda i: (i, 0)),
            pl.BlockSpec((C, C), lambda i: (0, 0)),
            pl.BlockSpec((C,), lambda i: (0,)),
        ],
        out_specs=pl.BlockSpec((blk, C), lambda i: (i, 0)),
        out_shape=jax.ShapeDtypeStruct((NPIX, C), jnp.float32),
    )(dcn, out_wt, out_b)


ROWS_PER_TEC = 14
ITEMS_PER_BLK = 8
NBLK = (H * G) // ITEMS_PER_BLK


def _sc_body(proj_hbm, offx_hbm, offy_hbm, m_hbm, inb_hbm, out_hbm,
             ox_v, oy_v, mm_v, inb_v, idx_v, rows_v, out_v, sem):
    wid = lax.axis_index("s") * 2 + lax.axis_index("c")
    pb = wid // 8
    rk = wid % 8
    pr = pb >> 1
    pc = pb & 1
    p16 = lax.iota(jnp.int32, 16)
    dyv = ((p16 & 3) - 1).astype(jnp.float32)
    dxv = ((p16 >> 2) - 1).astype(jnp.float32)
    pltpu.sync_copy(inb_hbm, inb_v)

    def axis_terms(pos):
        ii = pos.astype(jnp.int32)
        ii = ii - jnp.where(pos < ii.astype(jnp.float32), 1, 0)
        fr = pos - ii.astype(jnp.float32)
        odd = ii & 1
        ie = ii + odd
        wl = jnp.where(odd == 0, 1.0 - fr, fr)
        vl = (ie >= 2) & (ie <= HO)
        s0 = jnp.where((ii >= 0) & (ii <= 226), 1.0 - fr, 0.0)
        s1 = jnp.where((ii >= -1) & (ii <= 225), fr, 0.0)
        return ie, wl, vl, s0 + s1

    def row_body(rr, carry):
        r = rk * ROWS_PER_TEC + rr
        fq0 = pb * NROW + r * H
        pltpu.sync_copy(offx_hbm.at[pl.ds(fq0, H)], ox_v)
        pltpu.sync_copy(offy_hbm.at[pl.ds(fq0, H)], oy_v)
        pltpu.sync_copy(m_hbm.at[pl.ds(fq0, H)], mm_v)
        hof = (2 * r + pr + 2).astype(jnp.float32)
        basey = hof - dyv

        def blk_body(b, c2):
            mws = []
            sbs = []
            for u in range(ITEMS_PER_BLK):
                t = b * ITEMS_PER_BLK + u
                s = t >> 2
                g = t & 3
                offx = ox_v[s, pl.ds(g * GC, GC)]
                offy = oy_v[s, pl.ds(g * GC, GC)]
                mv = mm_v[s, pl.ds(g * GC, GC)]
                wof = (2 * s + pc + 2).astype(jnp.float32)
                py = basey - offy
                px = (wof - dxv) - offx
                ye, wy, vy, sy = axis_terms(py)
                xe, wx, vx, sx = axis_terms(px)
                valid = vy & vx
                w = jnp.where(valid, wy * wx, 0.0)
                row = ((ye - 2) >> 1) * H + ((xe - 2) >> 1)
                row = jnp.where(valid, row, 0) + g * NROW
                idx_v[pl.ds(u * P, P)] = row
                mws.append(mv * w)
                msv = mv * (sy * sx)
                lanes = [msv[i] for i in range(P)]
                while len(lanes) > 1:
                    lanes = [lanes[i] + lanes[i + 1] for i in range(0, len(lanes), 2)]
                sbs.append(lanes[0])
            pltpu.async_copy(proj_hbm.at[idx_v], rows_v, sem).wait()
            for u in range(ITEMS_PER_BLK):
                t = b * ITEMS_PER_BLK + u
                s = t >> 2
                g = t & 3
                acc = inb_v[pl.ds(g * GC, GC)] * sbs[u]
                for p in range(P):
                    acc = acc + rows_v[u * P + p, :] * mws[u][p]
                out_v[s, pl.ds(g * GC, GC)] = acc
            return c2

        lax.fori_loop(0, NBLK, blk_body, 0)
        pltpu.sync_copy(out_v, out_hbm.at[pl.ds(fq0, H)])
        return carry

    lax.fori_loop(0, ROWS_PER_TEC, row_body, 0)


def _sc_call(table, offx, offy, m, in_b):
    mesh = plsc.VectorSubcoreMesh(core_axis_name="c", subcore_axis_name="s")
    f = pl.kernel(
        _sc_body,
        out_type=jax.ShapeDtypeStruct((NPIX, C), jnp.float32),
        mesh=mesh,
        scratch_types=[
            pltpu.VMEM((H, C), jnp.float32),
            pltpu.VMEM((H, C), jnp.float32),
            pltpu.VMEM((H, C), jnp.float32),
            pltpu.VMEM((C,), jnp.float32),
            pltpu.VMEM((ITEMS_PER_BLK * P,), jnp.int32),
            pltpu.VMEM((ITEMS_PER_BLK * P, GC), jnp.float32),
            pltpu.VMEM((H, C), jnp.float32),
            pltpu.SemaphoreType.DMA,
        ],
        compiler_params=pltpu.CompilerParams(use_tc_tiling_on_sc=False),
    )
    return f(table, offx, offy, m, in_b)


def kernel(input, dw_w, dw_b, ln_w, ln_b, off_w, off_b, mask_w, mask_b,
           in_w, in_b, out_w, out_b):
    inp = input[0]
    inpF = jnp.flip(inp, (0, 1))
    inpad = jnp.pad(inpF, ((1, 1), (1, 1), (0, 0)))

    dwk = dw_w[:, 0]
    wcoef = jnp.stack([
        jnp.stack([dwk[:, int(2 * a + 1 - (pb // 2)), int(2 * b + 1 - (pb % 2))]
                   for a in (0, 1) for b in (0, 1)], axis=0)
        for pb in range(4)
    ], axis=0)

    wox = off_w[0::2].T
    woy = off_w[1::2].T
    wm = mask_w.T
    box = off_b[0::2]
    boy = off_b[1::2]
    bm = mask_b
    gid = jnp.arange(C) // GC
    bd = (gid[:, None] == gid[None, :]).astype(jnp.float32)

    proj = _proj_call(inp.reshape(NROW, C), in_w.T)
    table = proj.reshape(NROW, G, GC).transpose(1, 0, 2).reshape(G * NROW, GC)

    offx, offy, m = _fields_call(inpad, wcoef, dw_b, ln_w, ln_b,
                                 wox, woy, wm, box, boy, bm, bd)

    dcn = _sc_call(table, offx, offy, m, in_b)

    outf = _out_call(dcn, out_w.T, out_b)
    out = outf.reshape(2, 2, H, H, C).transpose(2, 0, 3, 1, 4).reshape(1, HO, HO, C)
    return out


# speedup vs baseline: 267.2538x; 1.4135x over previous
import functools
import math

import jax
import jax.numpy as jnp
from jax import lax
from jax.experimental import pallas as pl
from jax.experimental.pallas import tpu as pltpu
from jax.experimental.pallas import tpu_sc as plsc

C = 64
G = 4
GC = 16
P = 16
H = 112
HO = 224
NPIX = HO * HO
NROW = H * H

def _proj_body(inp_ref, wt_ref, out_ref):
    out_ref[:] = jnp.dot(inp_ref[:], wt_ref[:], preferred_element_type=jnp.float32)


def _proj_call(inp_flat, in_wt):
    return pl.pallas_call(
        _proj_body,
        out_shape=jax.ShapeDtypeStruct((NROW, C), jnp.float32),
    )(inp_flat, in_wt)


TR = 28
NRC = H // TR


def _fields_body(inpad_ref, wcoef_ref, dwb_ref, lnw_ref, lnb_ref,
                 wox_ref, woy_ref, wm_ref, box_ref, boy_ref, bm_ref, bd_ref,
                 ox_ref, oy_ref, m_ref):
    pb = pl.program_id(0)
    rc = pl.program_id(1)
    pr = pb // 2
    pc = pb % 2
    r0 = rc * TR
    x = jnp.zeros((TR, H, C), jnp.float32)
    for ta in range(2):
        for tb in range(2):
            w = wcoef_ref[pb, ta * 2 + tb, :]
            rs = r0 + 1 + pr - ta
            cs = 1 + pc - tb
            sl = inpad_ref[pl.ds(rs, TR), pl.ds(cs, H), :]
            x = x + sl * w[None, None, :]
    x = x.reshape(TR * H, C) + dwb_ref[:]
    mu = jnp.mean(x, -1, keepdims=True)
    var = jnp.mean((x - mu) * (x - mu), -1, keepdims=True)
    x = (x - mu) * lax.rsqrt(var + 1e-6) * lnw_ref[:] + lnb_ref[:]
    x = 0.5 * x * (1.0 + lax.erf(x * (1.0 / math.sqrt(2.0))))
    ox_ref[:] = jnp.dot(x, wox_ref[:], preferred_element_type=jnp.float32) + box_ref[:]
    oy_ref[:] = jnp.dot(x, woy_ref[:], preferred_element_type=jnp.float32) + boy_ref[:]
    ml = jnp.dot(x, wm_ref[:], preferred_element_type=jnp.float32) + bm_ref[:]
    ml = ml - jnp.max(ml, -1, keepdims=True)
    e = jnp.exp(ml)
    ssum = jnp.dot(e, bd_ref[:], preferred_element_type=jnp.float32)
    m_ref[:] = e / ssum


def _fields_call(inpad, wcoef, dw_b, ln_w, ln_b, wox, woy, wm, box, boy, bm, bd):
    blk = TR * H
    full = lambda shp: pl.BlockSpec(shp, lambda pb, rc: (0,) * len(shp))
    out_spec = pl.BlockSpec((blk, C), lambda pb, rc: (pb * NRC + rc, 0))
    out = jax.ShapeDtypeStruct((NPIX, C), jnp.float32)
    return pl.pallas_call(
        _fields_body,
        grid=(4, NRC),
        in_specs=[
            full((H + 2, H + 2, C)),
            full((4, 4, C)),
            full((C,)), full((C,)), full((C,)),
            full((C, C)), full((C, C)), full((C, C)),
            full((C,)), full((C,)), full((C,)),
            full((C, C)),
        ],
        out_specs=[out_spec, out_spec, out_spec],
        out_shape=[out, out, out],
    )(inpad, wcoef, dw_b, ln_w, ln_b, wox, woy, wm, box, boy, bm, bd)


def _out_body(inp_ref, wt_ref, b_ref, out_ref):
    out_ref[:] = jnp.dot(inp_ref[:], wt_ref[:], preferred_element_type=jnp.float32) + b_ref[:]


def _out_call(dcn, out_wt, out_b):
    blk = NPIX // 16
    return pl.pallas_call(
        _out_body,
        grid=(16,),
        in_specs=[
            pl.BlockSpec((blk, C), lambda i: (i, 0)),
            pl.BlockSpec((C, C), lambda i: (0, 0)),
            pl.BlockSpec((C,), lambda i: (0,)),
        ],
        out_specs=pl.BlockSpec((blk, C), lambda i: (i, 0)),
        out_shape=jax.ShapeDtypeStruct((NPIX, C), jnp.float32),
    )(dcn, out_wt, out_b)


ROWS_PER_TEC = 14
ITEMS_PER_BLK = 8
NBLK = (H * G) // ITEMS_PER_BLK


def _sc_body(proj_hbm, offx_hbm, offy_hbm, m_hbm, inb_hbm, out_hbm,
             ox_v, oy_v, mm_v, inb_v, idx_a, idx_b, rows_a, rows_b, out_v,
             sem_a, sem_b):
    wid = lax.axis_index("s") * 2 + lax.axis_index("c")
    pb = wid // 8
    rk = wid % 8
    pr = pb >> 1
    pc = pb & 1
    p16 = lax.iota(jnp.int32, 16)
    dyv = ((p16 & 3) - 1).astype(jnp.float32)
    dxv = ((p16 >> 2) - 1).astype(jnp.float32)
    pltpu.sync_copy(inb_hbm, inb_v)

    def axis_terms(pos):
        ii = pos.astype(jnp.int32)
        ii = ii - jnp.where(pos < ii.astype(jnp.float32), 1, 0)
        fr = pos - ii.astype(jnp.float32)
        odd = ii & 1
        ie = ii + odd
        wl = jnp.where(odd == 0, 1.0 - fr, fr)
        vl = (ie >= 2) & (ie <= HO)
        s0 = jnp.where((ii >= 0) & (ii <= 226), 1.0 - fr, 0.0)
        s1 = jnp.where((ii >= -1) & (ii <= 225), fr, 0.0)
        return ie, wl, vl, s0 + s1

    def row_body(rr, carry):
        r = rk * ROWS_PER_TEC + rr
        fq0 = pb * NROW + r * H
        pltpu.sync_copy(offx_hbm.at[pl.ds(fq0, H)], ox_v)
        pltpu.sync_copy(offy_hbm.at[pl.ds(fq0, H)], oy_v)
        pltpu.sync_copy(m_hbm.at[pl.ds(fq0, H)], mm_v)
        hof = (2 * r + pr + 2).astype(jnp.float32)
        basey = hof - dyv

        def compute_fire(b, idx_v, rows_v, sem):
            mws = []
            sbs = []
            for u in range(ITEMS_PER_BLK):
                t = b * ITEMS_PER_BLK + u
                s = t >> 2
                g = t & 3
                offx = ox_v[s, pl.ds(g * GC, GC)]
                offy = oy_v[s, pl.ds(g * GC, GC)]
                mv = mm_v[s, pl.ds(g * GC, GC)]
                wof = (2 * s + pc + 2).astype(jnp.float32)
                py = basey - offy
                px = (wof - dxv) - offx
                ye, wy, vy, sy = axis_terms(py)
                xe, wx, vx, sx = axis_terms(px)
                valid = vy & vx
                w = jnp.where(valid, wy * wx, 0.0)
                row = ((ye - 2) >> 1) * H + ((xe - 2) >> 1)
                row = jnp.where(valid, row, 0) + g * NROW
                idx_v[pl.ds(u * P, P)] = row
                mws.append(mv * w)
                msv = mv * (sy * sx)
                lanes = [msv[i] for i in range(P)]
                while len(lanes) > 1:
                    lanes = [lanes[i] + lanes[i + 1] for i in range(0, len(lanes), 2)]
                sbs.append(lanes[0])
            pltpu.async_copy(proj_hbm.at[idx_v], rows_v, sem)
            return tuple(mws), tuple(sbs)

        def accumulate(b, idx_v, rows_v, sem, wts):
            mws, sbs = wts
            pltpu.make_async_copy(proj_hbm.at[idx_v], rows_v, sem).wait()
            for u in range(ITEMS_PER_BLK):
                t = b * ITEMS_PER_BLK + u
                s = t >> 2
                g = t & 3
                acc = inb_v[pl.ds(g * GC, GC)] * sbs[u]
                for p in range(P):
                    acc = acc + rows_v[u * P + p, :] * mws[u][p]
                out_v[s, pl.ds(g * GC, GC)] = acc

        def pair_body(k, wts_even):
            b = 2 * k
            wts_odd = compute_fire(b + 1, idx_b, rows_b, sem_b)
            accumulate(b, idx_a, rows_a, sem_a, wts_even)
            wts_next = compute_fire(b + 2, idx_a, rows_a, sem_a)
            accumulate(b + 1, idx_b, rows_b, sem_b, wts_odd)
            return wts_next

        wts0 = compute_fire(0, idx_a, rows_a, sem_a)
        wts_last = lax.fori_loop(0, NBLK // 2 - 1, pair_body, wts0)
        wts_odd = compute_fire(NBLK - 1, idx_b, rows_b, sem_b)
        accumulate(NBLK - 2, idx_a, rows_a, sem_a, wts_last)
        accumulate(NBLK - 1, idx_b, rows_b, sem_b, wts_odd)
        pltpu.sync_copy(out_v, out_hbm.at[pl.ds(fq0, H)])
        return carry

    lax.fori_loop(0, ROWS_PER_TEC, row_body, 0)


def _sc_call(table, offx, offy, m, in_b):
    mesh = plsc.VectorSubcoreMesh(core_axis_name="c", subcore_axis_name="s")
    f = pl.kernel(
        _sc_body,
        out_type=jax.ShapeDtypeStruct((NPIX, C), jnp.float32),
        mesh=mesh,
        scratch_types=[
            pltpu.VMEM((H, C), jnp.float32),
            pltpu.VMEM((H, C), jnp.float32),
            pltpu.VMEM((H, C), jnp.float32),
            pltpu.VMEM((C,), jnp.float32),
            pltpu.VMEM((ITEMS_PER_BLK * P,), jnp.int32),
            pltpu.VMEM((ITEMS_PER_BLK * P,), jnp.int32),
            pltpu.VMEM((ITEMS_PER_BLK * P, GC), jnp.float32),
            pltpu.VMEM((ITEMS_PER_BLK * P, GC), jnp.float32),
            pltpu.VMEM((H, C), jnp.float32),
            pltpu.SemaphoreType.DMA,
            pltpu.SemaphoreType.DMA,
        ],
        compiler_params=pltpu.CompilerParams(use_tc_tiling_on_sc=False),
    )
    return f(table, offx, offy, m, in_b)


def kernel(input, dw_w, dw_b, ln_w, ln_b, off_w, off_b, mask_w, mask_b,
           in_w, in_b, out_w, out_b):
    inp = input[0]
    inpF = jnp.flip(inp, (0, 1))
    inpad = jnp.pad(inpF, ((1, 1), (1, 1), (0, 0)))

    dwk = dw_w[:, 0]
    wcoef = jnp.stack([
        jnp.stack([dwk[:, int(2 * a + 1 - (pb // 2)), int(2 * b + 1 - (pb % 2))]
                   for a in (0, 1) for b in (0, 1)], axis=0)
        for pb in range(4)
    ], axis=0)

    wox = off_w[0::2].T
    woy = off_w[1::2].T
    wm = mask_w.T
    box = off_b[0::2]
    boy = off_b[1::2]
    bm = mask_b
    gid = jnp.arange(C) // GC
    bd = (gid[:, None] == gid[None, :]).astype(jnp.float32)

    proj = _proj_call(inp.reshape(NROW, C), in_w.T)
    table = proj.reshape(NROW, G, GC).transpose(1, 0, 2).reshape(G * NROW, GC)

    offx, offy, m = _fields_call(inpad, wcoef, dw_b, ln_w, ln_b,
                                 wox, woy, wm, box, boy, bm, bd)

    dcn = _sc_call(table, offx, offy, m, in_b)

    outf = _out_call(dcn, out_w.T, out_b)
    out = outf.reshape(2, 2, H, H, C).transpose(2, 0, 3, 1, 4).reshape(1, HO, HO, C)
    return out


# speedup vs baseline: 350.5126x; 1.3115x over previous
import functools
import math

import jax
import jax.numpy as jnp
from jax import lax
from jax.experimental import pallas as pl
from jax.experimental.pallas import tpu as pltpu
from jax.experimental.pallas import tpu_sc as plsc

C = 64
G = 4
GC = 16
P = 16
H = 112
HO = 224
NPIX = HO * HO
NROW = H * H

def _proj_body(inp_ref, wt_ref, out_ref):
    out_ref[:] = jnp.dot(inp_ref[:], wt_ref[:], preferred_element_type=jnp.float32)


def _proj_call(inp_flat, in_wt):
    return pl.pallas_call(
        _proj_body,
        out_shape=jax.ShapeDtypeStruct((NROW, C), jnp.float32),
    )(inp_flat, in_wt)


TR = 28
NRC = H // TR


def _axis_terms_tc(pos):
    i0 = jnp.floor(pos)
    fr = pos - i0
    ii = i0.astype(jnp.int32)
    odd = ii & 1
    ie = ii + odd
    wl = jnp.where(odd == 0, 1.0 - fr, fr)
    vl = (ie >= 2) & (ie <= HO)
    S = jnp.where((ii >= 0) & (ii <= 226), 1.0 - fr, 0.0) + \
        jnp.where((ii >= -1) & (ii <= 225), fr, 0.0)
    return ie, wl, vl, S


def _fields_body(inpad_ref, wcoef_ref, dwb_ref, lnw_ref, lnb_ref,
                 wox_ref, woy_ref, wm_ref, box_ref, boy_ref, bm_ref,
                 bd_ref, bd4_ref, dxc_ref, dyc_ref, gofs_ref,
                 idx_ref, mw_ref, sb_ref):
    pb = pl.program_id(0)
    rc = pl.program_id(1)
    pr = pb // 2
    pc = pb % 2
    r0 = rc * TR
    x = jnp.zeros((TR, H, C), jnp.float32)
    for ta in range(2):
        for tb in range(2):
            w = wcoef_ref[pb, ta * 2 + tb, :]
            rs = r0 + 1 + pr - ta
            cs = 1 + pc - tb
            sl = inpad_ref[pl.ds(rs, TR), pl.ds(cs, H), :]
            x = x + sl * w[None, None, :]
    x = x.reshape(TR * H, C) + dwb_ref[:]
    mu = jnp.mean(x, -1, keepdims=True)
    var = jnp.mean((x - mu) * (x - mu), -1, keepdims=True)
    x = (x - mu) * lax.rsqrt(var + 1e-6) * lnw_ref[:] + lnb_ref[:]
    x = 0.5 * x * (1.0 + lax.erf(x * (1.0 / math.sqrt(2.0))))
    offx = jnp.dot(x, wox_ref[:], preferred_element_type=jnp.float32) + box_ref[:]
    offy = jnp.dot(x, woy_ref[:], preferred_element_type=jnp.float32) + boy_ref[:]
    ml = jnp.dot(x, wm_ref[:], preferred_element_type=jnp.float32) + bm_ref[:]
    ml = ml - jnp.max(ml, -1, keepdims=True)
    e = jnp.exp(ml)
    m = e / jnp.dot(e, bd_ref[:], preferred_element_type=jnp.float32)

    blk = TR * H
    ii = lax.broadcasted_iota(jnp.int32, (blk, 1), 0)
    ho = 2 * (r0 + ii // H) + pr
    wo = 2 * (ii % H) + pc
    py = (ho + 2).astype(jnp.float32) - dyc_ref[:][None, :] - offy
    px = (wo + 2).astype(jnp.float32) - dxc_ref[:][None, :] - offx
    ye, wy, vy, sy = _axis_terms_tc(py)
    xe, wx, vx, sx = _axis_terms_tc(px)
    valid = vy & vx
    w = jnp.where(valid, wy * wx, 0.0)
    rowi = ((ye - 2) >> 1) * H + ((xe - 2) >> 1)
    idx_ref[:] = jnp.where(valid, rowi, 0) + gofs_ref[:][None, :]
    mw_ref[:] = m * w
    sb_ref[:] = jnp.dot(m * (sy * sx), bd4_ref[:], preferred_element_type=jnp.float32)


def _fields_call(inpad, wcoef, dw_b, ln_w, ln_b, wox, woy, wm, box, boy, bm,
                 bd, bd4, dxc, dyc, gofs):
    blk = TR * H
    full = lambda shp: pl.BlockSpec(shp, lambda pb, rc: (0,) * len(shp))
    out_spec = pl.BlockSpec((blk, C), lambda pb, rc: (pb * NRC + rc, 0))
    return pl.pallas_call(
        _fields_body,
        grid=(4, NRC),
        in_specs=[
            full((H + 2, H + 2, C)),
            full((4, 4, C)),
            full((C,)), full((C,)), full((C,)),
            full((C, C)), full((C, C)), full((C, C)),
            full((C,)), full((C,)), full((C,)),
            full((C, C)), full((C, G)),
            full((C,)), full((C,)), full((C,)),
        ],
        out_specs=[
            out_spec, out_spec,
            pl.BlockSpec((blk, G), lambda pb, rc: (pb * NRC + rc, 0)),
        ],
        out_shape=[
            jax.ShapeDtypeStruct((NPIX, C), jnp.int32),
            jax.ShapeDtypeStruct((NPIX, C), jnp.float32),
            jax.ShapeDtypeStruct((NPIX, G), jnp.float32),
        ],
    )(inpad, wcoef, dw_b, ln_w, ln_b, wox, woy, wm, box, boy, bm,
      bd, bd4, dxc, dyc, gofs)


def _out_body(inp_ref, sb_ref, e_ref, wt_ref, b_ref, out_ref):
    full = inp_ref[:] + jnp.dot(sb_ref[:], e_ref[:], preferred_element_type=jnp.float32)
    out_ref[:] = jnp.dot(full, wt_ref[:], preferred_element_type=jnp.float32) + b_ref[:]


def _out_call(dcn, sb, emat, out_wt, out_b):
    blk = NPIX // 16
    return pl.pallas_call(
        _out_body,
        grid=(16,),
        in_specs=[
            pl.BlockSpec((blk, C), lambda i: (i, 0)),
            pl.BlockSpec((blk, G), lambda i: (i, 0)),
            pl.BlockSpec((G, C), lambda i: (0, 0)),
            pl.BlockSpec((C, C), lambda i: (0, 0)),
            pl.BlockSpec((C,), lambda i: (0,)),
        ],
        out_specs=pl.BlockSpec((blk, C), lambda i: (i, 0)),
        out_shape=jax.ShapeDtypeStruct((NPIX, C), jnp.float32),
    )(dcn, sb, emat, out_wt, out_b)


ROWS_PER_TEC = 14
QS = 28


def _sc_body(proj_hbm, idx_hbm, mw_hbm, out_hbm,
             idx_v, mw_v, rows_a, rows_b, out_v, sem_a, sem_b):
    wid = lax.axis_index("s") * 2 + lax.axis_index("c")
    pb = wid // 8
    rk = wid % 8
    p16 = lax.iota(jnp.int32, 16)
    cps = [p16 * 0 + p for p in range(P)]

    def bcast(v, p):
        return v.at[cps[p]].get(mode="promise_in_bounds")

    def row_body(rr, carry):
        r = rk * ROWS_PER_TEC + rr
        fq0 = pb * NROW + r * H
        grow = pb * H + r
        pltpu.sync_copy(idx_hbm.at[grow], idx_v)
        pltpu.sync_copy(mw_hbm.at[pl.ds(fq0, H)], mw_v)
        pltpu.async_copy(proj_hbm.at[idx_v.at[0]], rows_a, sem_a)
        for q in range(4):
            rows_q, sem_q = (rows_a, sem_a) if q % 2 == 0 else (rows_b, sem_b)
            if q < 3:
                rows_n, sem_n = (rows_b, sem_b) if q % 2 == 0 else (rows_a, sem_a)
                pltpu.async_copy(proj_hbm.at[idx_v.at[q + 1]], rows_n, sem_n)
            pltpu.make_async_copy(
                proj_hbm.at[idx_v.at[q]], rows_q, sem_q).wait()

            def sq_body(sq, c2, rows_q=rows_q, q=q):
                s = q * QS + sq
                base = sq * C
                for g in range(G):
                    mwv = mw_v[s, pl.ds(g * GC, GC)]
                    acc = rows_q[base + g * GC, :] * bcast(mwv, 0)
                    for p in range(1, P):
                        acc = acc + rows_q[base + g * GC + p, :] * bcast(mwv, p)
                    out_v[s, pl.ds(g * GC, GC)] = acc
                return c2

            lax.fori_loop(0, QS, sq_body, 0)
        pltpu.sync_copy(out_v, out_hbm.at[pl.ds(fq0, H)])
        return carry

    lax.fori_loop(0, ROWS_PER_TEC, row_body, 0)


def _sc_call(table, idx, mw):
    mesh = plsc.VectorSubcoreMesh(core_axis_name="c", subcore_axis_name="s")
    f = pl.kernel(
        _sc_body,
        out_type=jax.ShapeDtypeStruct((NPIX, C), jnp.float32),
        mesh=mesh,
        scratch_types=[
            pltpu.VMEM((4, QS * C), jnp.int32),
            pltpu.VMEM((H, C), jnp.float32),
            pltpu.VMEM((QS * C, GC), jnp.float32),
            pltpu.VMEM((QS * C, GC), jnp.float32),
            pltpu.VMEM((H, C), jnp.float32),
            pltpu.SemaphoreType.DMA,
            pltpu.SemaphoreType.DMA,
        ],
        compiler_params=pltpu.CompilerParams(use_tc_tiling_on_sc=False),
    )
    return f(table, idx, mw)


def kernel(input, dw_w, dw_b, ln_w, ln_b, off_w, off_b, mask_w, mask_b,
           in_w, in_b, out_w, out_b):
    inp = input[0]
    inpF = jnp.flip(inp, (0, 1))
    inpad = jnp.pad(inpF, ((1, 1), (1, 1), (0, 0)))

    dwk = dw_w[:, 0]
    wcoef = jnp.stack([
        jnp.stack([dwk[:, int(2 * a + 1 - (pb // 2)), int(2 * b + 1 - (pb % 2))]
                   for a in (0, 1) for b in (0, 1)], axis=0)
        for pb in range(4)
    ], axis=0)

    wox = off_w[0::2].T
    woy = off_w[1::2].T
    wm = mask_w.T
    box = off_b[0::2]
    boy = off_b[1::2]
    bm = mask_b
    gid = jnp.arange(C) // GC
    bd = (gid[:, None] == gid[None, :]).astype(jnp.float32)
    bd4 = (gid[:, None] == jnp.arange(G)[None, :]).astype(jnp.float32)
    pch = jnp.arange(C) % P
    dxc = ((pch // 4) - 1).astype(jnp.float32)
    dyc = ((pch % 4) - 1).astype(jnp.float32)
    gofs = (jnp.arange(C) // GC * NROW).astype(jnp.int32)
    emat = (jnp.arange(G)[:, None] == gid[None, :]).astype(jnp.float32) * in_b[None, :]

    proj = _proj_call(inp.reshape(NROW, C), in_w.T)
    table = proj.reshape(NROW, G, GC).transpose(1, 0, 2).reshape(G * NROW, GC)

    idx, mw, sb = _fields_call(inpad, wcoef, dw_b, ln_w, ln_b,
                               wox, woy, wm, box, boy, bm,
                               bd, bd4, dxc, dyc, gofs)

    dcn = _sc_call(table, idx.reshape(2 * HO, 4, QS * C), mw)

    outf = _out_call(dcn, sb, emat, out_w.T, out_b)
    out = outf.reshape(2, 2, H, H, C).transpose(2, 0, 3, 1, 4).reshape(1, HO, HO, C)
    return out


# speedup vs baseline: 397.2731x; 1.1334x over previous
import functools
import math

import jax
import jax.numpy as jnp
from jax import lax
from jax.experimental import pallas as pl
from jax.experimental.pallas import tpu as pltpu
from jax.experimental.pallas import tpu_sc as plsc

C = 64
G = 4
GC = 16
P = 16
H = 112
HO = 224
NPIX = HO * HO
NROW = H * H

def _proj_body(inp_ref, wt_ref, out_ref):
    out_ref[0] = jnp.dot(inp_ref[:], wt_ref[0], preferred_element_type=jnp.float32)


def _proj_call(inp_flat, in_wt):
    return pl.pallas_call(
        _proj_body,
        grid=(G,),
        in_specs=[
            pl.BlockSpec((NROW, C), lambda g: (0, 0)),
            pl.BlockSpec((1, C, GC), lambda g: (g, 0, 0)),
        ],
        out_specs=pl.BlockSpec((1, NROW, GC), lambda g: (g, 0, 0)),
        out_shape=jax.ShapeDtypeStruct((G, NROW, GC), jnp.float32),
    )(inp_flat, in_wt)


TR = 28
NRC = H // TR


def _axis_terms_tc(pos):
    i0 = jnp.floor(pos)
    fr = pos - i0
    ii = i0.astype(jnp.int32)
    odd = ii & 1
    ie = ii + odd
    wl = jnp.where(odd == 0, 1.0 - fr, fr)
    vl = (ie >= 2) & (ie <= HO)
    S = jnp.where((ii >= 0) & (ii <= 226), 1.0 - fr, 0.0) + \
        jnp.where((ii >= -1) & (ii <= 225), fr, 0.0)
    return ie, wl, vl, S


def _fields_body(inpad_ref, wcoef_ref, dwb_ref, lnw_ref, lnb_ref,
                 wox_ref, woy_ref, wm_ref, box_ref, boy_ref, bm_ref,
                 bd_ref, bd4_ref, dxc_ref, dyc_ref, gofs_ref,
                 idx_ref, mw_ref, sb_ref):
    pb = pl.program_id(0)
    rc = pl.program_id(1)
    pr = pb // 2
    pc = pb % 2
    r0 = rc * TR
    x = jnp.zeros((TR, H, C), jnp.float32)
    for ta in range(2):
        for tb in range(2):
            w = wcoef_ref[pb, ta * 2 + tb, :]
            rs = r0 + ta + pr
            cs = tb + pc
            sl = inpad_ref[pl.ds(rs, TR), pl.ds(cs, H), :]
            x = x + sl * w[None, None, :]
    x = x.reshape(TR * H, C) + dwb_ref[:]
    mu = jnp.mean(x, -1, keepdims=True)
    var = jnp.mean((x - mu) * (x - mu), -1, keepdims=True)
    x = (x - mu) * lax.rsqrt(var + 1e-6) * lnw_ref[:] + lnb_ref[:]
    x = 0.5 * x * (1.0 + lax.erf(x * (1.0 / math.sqrt(2.0))))
    offx = jnp.dot(x, wox_ref[:], preferred_element_type=jnp.float32) + box_ref[:]
    offy = jnp.dot(x, woy_ref[:], preferred_element_type=jnp.float32) + boy_ref[:]
    ml = jnp.dot(x, wm_ref[:], preferred_element_type=jnp.float32) + bm_ref[:]
    ml = ml - jnp.max(ml, -1, keepdims=True)
    e = jnp.exp(ml)
    m = e / jnp.dot(e, bd_ref[:], preferred_element_type=jnp.float32)

    blk = TR * H
    ii = lax.broadcasted_iota(jnp.int32, (blk, 1), 0)
    ho = (HO - 1) - (2 * (r0 + ii // H) + pr)
    wo = (HO - 1) - (2 * (ii % H) + pc)
    py = (ho + 2).astype(jnp.float32) - dyc_ref[:][None, :] - offy
    px = (wo + 2).astype(jnp.float32) - dxc_ref[:][None, :] - offx
    ye, wy, vy, sy = _axis_terms_tc(py)
    xe, wx, vx, sx = _axis_terms_tc(px)
    valid = vy & vx
    w = jnp.where(valid, wy * wx, 0.0)
    rowi = ((ye - 2) >> 1) * H + ((xe - 2) >> 1)
    idx_ref[:] = jnp.where(valid, rowi, 0) + gofs_ref[:][None, :]
    mw_ref[:] = m * w
    sb_ref[:] = jnp.dot(m * (sy * sx), bd4_ref[:], preferred_element_type=jnp.float32)


def _fields_call(inpad, wcoef, dw_b, ln_w, ln_b, wox, woy, wm, box, boy, bm,
                 bd, bd4, dxc, dyc, gofs):
    blk = TR * H
    full = lambda shp: pl.BlockSpec(shp, lambda pb, rc: (0,) * len(shp))
    out_spec = pl.BlockSpec((blk, C), lambda pb, rc: (pb * NRC + rc, 0))
    return pl.pallas_call(
        _fields_body,
        grid=(4, NRC),
        in_specs=[
            full((H + 2, H + 2, C)),
            full((4, 4, C)),
            full((C,)), full((C,)), full((C,)),
            full((C, C)), full((C, C)), full((C, C)),
            full((C,)), full((C,)), full((C,)),
            full((C, C)), full((C, G)),
            full((C,)), full((C,)), full((C,)),
        ],
        out_specs=[
            out_spec, out_spec,
            pl.BlockSpec((blk, G), lambda pb, rc: (pb * NRC + rc, 0)),
        ],
        out_shape=[
            jax.ShapeDtypeStruct((NPIX, C), jnp.int32),
            jax.ShapeDtypeStruct((NPIX, C), jnp.float32),
            jax.ShapeDtypeStruct((NPIX, G), jnp.float32),
        ],
    )(inpad, wcoef, dw_b, ln_w, ln_b, wox, woy, wm, box, boy, bm,
      bd, bd4, dxc, dyc, gofs)


def _out_body(inp_ref, sb_ref, e_ref, wt_ref, b_ref, out_ref):
    full = inp_ref[:] + jnp.dot(sb_ref[:], e_ref[:], preferred_element_type=jnp.float32)
    out_ref[:] = jnp.dot(full, wt_ref[:], preferred_element_type=jnp.float32) + b_ref[:]


def _out_call(dcn, sb, emat, out_wt, out_b):
    blk = NPIX // 16
    return pl.pallas_call(
        _out_body,
        grid=(16,),
        in_specs=[
            pl.BlockSpec((blk, C), lambda i: (i, 0)),
            pl.BlockSpec((blk, G), lambda i: (i, 0)),
            pl.BlockSpec((G, C), lambda i: (0, 0)),
            pl.BlockSpec((C, C), lambda i: (0, 0)),
            pl.BlockSpec((C,), lambda i: (0,)),
        ],
        out_specs=pl.BlockSpec((blk, C), lambda i: (i, 0)),
        out_shape=jax.ShapeDtypeStruct((NPIX, C), jnp.float32),
    )(dcn, sb, emat, out_wt, out_b)


ROWS_PER_TEC = 14
QS = 28


def _sc_body(proj_hbm, idx_hbm, mw_hbm, out_hbm,
             idx_v, mw_v, rows_a, rows_b, out_v, sem_a, sem_b):
    wid = lax.axis_index("s") * 2 + lax.axis_index("c")
    pb = wid // 8
    rk = wid % 8
    p16 = lax.iota(jnp.int32, 16)
    cps = [p16 * 0 + p for p in range(P)]

    def bcast(v, p):
        return v.at[cps[p]].get(mode="promise_in_bounds")

    def row_body(rr, carry):
        r = rk * ROWS_PER_TEC + rr
        fq0 = pb * NROW + r * H
        fq0_out = (3 - pb) * NROW + (H - 1 - r) * H
        grow = pb * H + r
        pltpu.sync_copy(idx_hbm.at[grow], idx_v)
        pltpu.sync_copy(mw_hbm.at[pl.ds(fq0, H)], mw_v)
        pltpu.async_copy(proj_hbm.at[idx_v.at[0]], rows_a, sem_a)
        for q in range(4):
            rows_q, sem_q = (rows_a, sem_a) if q % 2 == 0 else (rows_b, sem_b)
            if q < 3:
                rows_n, sem_n = (rows_b, sem_b) if q % 2 == 0 else (rows_a, sem_a)
                pltpu.async_copy(proj_hbm.at[idx_v.at[q + 1]], rows_n, sem_n)
            pltpu.make_async_copy(
                proj_hbm.at[idx_v.at[q]], rows_q, sem_q).wait()

            def sq_body(sq, c2, rows_q=rows_q, q=q):
                s = q * QS + sq
                base = sq * C
                for g in range(G):
                    mwv = mw_v[s, pl.ds(g * GC, GC)]
                    prods = [rows_q[base + g * GC + p, :] * bcast(mwv, p)
                             for p in range(P)]
                    while len(prods) > 1:
                        prods = [prods[i] + prods[i + 1]
                                 for i in range(0, len(prods), 2)]
                    out_v[H - 1 - s, pl.ds(g * GC, GC)] = prods[0]
                return c2

            lax.fori_loop(0, QS, sq_body, 0)
        pltpu.sync_copy(out_v, out_hbm.at[pl.ds(fq0_out, H)])
        return carry

    lax.fori_loop(0, ROWS_PER_TEC, row_body, 0)


def _sc_call(table, idx, mw):
    mesh = plsc.VectorSubcoreMesh(core_axis_name="c", subcore_axis_name="s")
    f = pl.kernel(
        _sc_body,
        out_type=jax.ShapeDtypeStruct((NPIX, C), jnp.float32),
        mesh=mesh,
        scratch_types=[
            pltpu.VMEM((4, QS * C), jnp.int32),
            pltpu.VMEM((H, C), jnp.float32),
            pltpu.VMEM((QS * C, GC), jnp.float32),
            pltpu.VMEM((QS * C, GC), jnp.float32),
            pltpu.VMEM((H, C), jnp.float32),
            pltpu.SemaphoreType.DMA,
            pltpu.SemaphoreType.DMA,
        ],
        compiler_params=pltpu.CompilerParams(use_tc_tiling_on_sc=False),
    )
    return f(table, idx, mw)


def kernel(input, dw_w, dw_b, ln_w, ln_b, off_w, off_b, mask_w, mask_b,
           in_w, in_b, out_w, out_b):
    inp = input[0]
    inpad = jnp.pad(inp, ((1, 1), (1, 1), (0, 0)))

    dwk = dw_w[:, 0]
    wcoef = jnp.stack([
        jnp.stack([dwk[:, int(2 * a + (pb // 2)), int(2 * b + (pb % 2))]
                   for a in (0, 1) for b in (0, 1)], axis=0)
        for pb in range(4)
    ], axis=0)

    wox = off_w[0::2].T
    woy = off_w[1::2].T
    wm = mask_w.T
    box = off_b[0::2]
    boy = off_b[1::2]
    bm = mask_b
    gid = jnp.arange(C) // GC
    bd = (gid[:, None] == gid[None, :]).astype(jnp.float32)
    bd4 = (gid[:, None] == jnp.arange(G)[None, :]).astype(jnp.float32)
    pch = jnp.arange(C) % P
    dxc = ((pch // 4) - 1).astype(jnp.float32)
    dyc = ((pch % 4) - 1).astype(jnp.float32)
    gofs = (jnp.arange(C) // GC * NROW).astype(jnp.int32)
    emat = (jnp.arange(G)[:, None] == gid[None, :]).astype(jnp.float32) * in_b[None, :]

    in_wt4 = in_w.T.reshape(C, G, GC).transpose(1, 0, 2)
    table = _proj_call(inp.reshape(NROW, C), in_wt4).reshape(G * NROW, GC)

    idx, mw, sb = _fields_call(inpad, wcoef, dw_b, ln_w, ln_b,
                               wox, woy, wm, box, boy, bm,
                               bd, bd4, dxc, dyc, gofs)

    dcn = _sc_call(table, idx.reshape(2 * HO, 4, QS * C), mw)

    sbf = jnp.flip(sb, 0)
    outf = _out_call(dcn, sbf, emat, out_w.T, out_b)
    out = outf.reshape(2, 2, H, H, C).transpose(2, 0, 3, 1, 4).reshape(1, HO, HO, C)
    return out


# speedup vs baseline: 413.3791x; 1.0405x over previous
import functools
import math

import jax
import jax.numpy as jnp
from jax import lax
from jax.experimental import pallas as pl
from jax.experimental.pallas import tpu as pltpu
from jax.experimental.pallas import tpu_sc as plsc

C = 64
G = 4
GC = 16
P = 16
H = 112
HO = 224
NPIX = HO * HO
NROW = H * H

def _proj_body(inp_ref, wt_ref, out_ref):
    out_ref[0] = jnp.dot(inp_ref[:], wt_ref[0], preferred_element_type=jnp.float32)


def _proj_call(inp_flat, in_wt):
    return pl.pallas_call(
        _proj_body,
        grid=(G,),
        in_specs=[
            pl.BlockSpec((NROW, C), lambda g: (0, 0)),
            pl.BlockSpec((1, C, GC), lambda g: (g, 0, 0)),
        ],
        out_specs=pl.BlockSpec((1, NROW, GC), lambda g: (g, 0, 0)),
        out_shape=jax.ShapeDtypeStruct((G, NROW, GC), jnp.float32),
    )(inp_flat, in_wt)


TR = 28
NRC = H // TR


def _axis_terms_tc(pos):
    i0 = jnp.floor(pos)
    fr = pos - i0
    ii = i0.astype(jnp.int32)
    odd = ii & 1
    ie = ii + odd
    wl = jnp.where(odd == 0, 1.0 - fr, fr)
    vl = (ie >= 2) & (ie <= HO)
    S = jnp.where((ii >= 0) & (ii <= 226), 1.0 - fr, 0.0) + \
        jnp.where((ii >= -1) & (ii <= 225), fr, 0.0)
    return ie, wl, vl, S


def _fields_body(inpad_ref, wcoef_ref, dwb_ref, lnw_ref, lnb_ref,
                 wox_ref, woy_ref, wm_ref, box_ref, boy_ref, bm_ref,
                 bd_ref, bd4_ref, dxc_ref, dyc_ref, gofs_ref,
                 idx_ref, mw_ref, sb_ref):
    pb = pl.program_id(0)
    rc = pl.program_id(1)
    pr = pb // 2
    pc = pb % 2
    r0 = rc * TR
    x = jnp.zeros((TR, H, C), jnp.float32)
    for ta in range(2):
        for tb in range(2):
            w = wcoef_ref[pb, ta * 2 + tb, :]
            rs = r0 + ta + pr
            cs = tb + pc
            sl = inpad_ref[pl.ds(rs, TR), pl.ds(cs, H), :]
            x = x + sl * w[None, None, :]
    x = x.reshape(TR * H, C) + dwb_ref[:]
    mu = jnp.mean(x, -1, keepdims=True)
    var = jnp.mean((x - mu) * (x - mu), -1, keepdims=True)
    x = (x - mu) * lax.rsqrt(var + 1e-6) * lnw_ref[:] + lnb_ref[:]
    x = 0.5 * x * (1.0 + lax.erf(x * (1.0 / math.sqrt(2.0))))
    offx = jnp.dot(x, wox_ref[:], preferred_element_type=jnp.float32) + box_ref[:]
    offy = jnp.dot(x, woy_ref[:], preferred_element_type=jnp.float32) + boy_ref[:]
    ml = jnp.dot(x, wm_ref[:], preferred_element_type=jnp.float32) + bm_ref[:]
    ml = ml - jnp.max(ml, -1, keepdims=True)
    e = jnp.exp(ml)
    m = e / jnp.dot(e, bd_ref[:], preferred_element_type=jnp.float32)

    blk = TR * H
    rI = lax.broadcasted_iota(jnp.int32, (TR, 1, 1), 0).astype(jnp.float32)
    sI = lax.broadcasted_iota(jnp.int32, (1, H, 1), 1).astype(jnp.float32)
    hof = ((HO + 1 - pr) - 2 * r0).astype(jnp.float32) - 2.0 * rI
    wof = (HO + 1 - pc).astype(jnp.float32) - 2.0 * sI
    py = (hof - dyc_ref[:][None, None, :]) - offy.reshape(TR, H, C)
    px = (wof - dxc_ref[:][None, None, :]) - offx.reshape(TR, H, C)
    ye, wy, vy, sy = _axis_terms_tc(py)
    xe, wx, vx, sx = _axis_terms_tc(px)
    valid = vy & vx
    w = jnp.where(valid, wy * wx, 0.0)
    rowi = ((ye - 2) >> 1) * H + ((xe - 2) >> 1)
    idxv = jnp.where(valid, rowi, 0).reshape(blk, C)
    idx_ref[:] = idxv + gofs_ref[:][None, :]
    mw_ref[:] = m * w.reshape(blk, C)
    sb_ref[:] = jnp.dot(m * (sy * sx).reshape(blk, C), bd4_ref[:],
                        preferred_element_type=jnp.float32)


def _fields_call(inpad, wcoef, dw_b, ln_w, ln_b, wox, woy, wm, box, boy, bm,
                 bd, bd4, dxc, dyc, gofs):
    blk = TR * H
    full = lambda shp: pl.BlockSpec(shp, lambda pb, rc: (0,) * len(shp))
    out_spec = pl.BlockSpec((blk, C), lambda pb, rc: (pb * NRC + rc, 0))
    return pl.pallas_call(
        _fields_body,
        grid=(4, NRC),
        in_specs=[
            full((H + 2, H + 2, C)),
            full((4, 4, C)),
            full((C,)), full((C,)), full((C,)),
            full((C, C)), full((C, C)), full((C, C)),
            full((C,)), full((C,)), full((C,)),
            full((C, C)), full((C, G)),
            full((C,)), full((C,)), full((C,)),
        ],
        out_specs=[
            out_spec, out_spec,
            pl.BlockSpec((blk, G), lambda pb, rc: (pb * NRC + rc, 0)),
        ],
        out_shape=[
            jax.ShapeDtypeStruct((NPIX, C), jnp.int32),
            jax.ShapeDtypeStruct((NPIX, C), jnp.float32),
            jax.ShapeDtypeStruct((NPIX, G), jnp.float32),
        ],
    )(inpad, wcoef, dw_b, ln_w, ln_b, wox, woy, wm, box, boy, bm,
      bd, bd4, dxc, dyc, gofs)


TRO = 28
NR4 = H // TRO


def _out_body(dcna_ref, dcnb_ref, wt_ref, b_ref, out_ref):
    oa = jnp.dot(dcna_ref[:], wt_ref[:], preferred_element_type=jnp.float32) + b_ref[:]
    ob = jnp.dot(dcnb_ref[:], wt_ref[:], preferred_element_type=jnp.float32) + b_ref[:]
    st = jnp.stack([oa.reshape(TRO, H, C), ob.reshape(TRO, H, C)], axis=2)
    out_ref[:] = st.reshape(TRO, 1, HO, C)


def _out_call(dcn, out_wt, out_b):
    blk = TRO * H
    return pl.pallas_call(
        _out_body,
        grid=(2, NR4),
        in_specs=[
            pl.BlockSpec((blk, C), lambda pr, rc: (2 * pr * NR4 + rc, 0)),
            pl.BlockSpec((blk, C), lambda pr, rc: ((2 * pr + 1) * NR4 + rc, 0)),
            pl.BlockSpec((C, C), lambda pr, rc: (0, 0)),
            pl.BlockSpec((C,), lambda pr, rc: (0,)),
        ],
        out_specs=pl.BlockSpec((TRO, 1, HO, C), lambda pr, rc: (rc, pr, 0, 0)),
        out_shape=jax.ShapeDtypeStruct((H, 2, HO, C), jnp.float32),
    )(dcn, dcn, out_wt, out_b)


ROWS_PER_TEC = 14
QS = 28


def _sc_body(proj_hbm, idx_hbm, mw_hbm, sb_hbm, inb_hbm, out_hbm,
             idx_v, mw_v, sb_v, inb_v, rows_a, rows_b, out_v, sem_a, sem_b):
    wid = lax.axis_index("s") * 2 + lax.axis_index("c")
    pb = wid // 8
    rk = wid % 8
    p16 = lax.iota(jnp.int32, 16)
    cps = [p16 * 0 + p for p in range(P)]
    z16 = p16 * 0
    pltpu.sync_copy(inb_hbm, inb_v)
    inb_gs = [inb_v[pl.ds(g * GC, GC)] for g in range(G)]

    def bcast(v, p):
        return v.at[cps[p]].get(mode="promise_in_bounds")

    def tree(vs):
        while len(vs) > 1:
            nxt = [vs[i] + vs[i + 1] for i in range(0, len(vs) - 1, 2)]
            if len(vs) % 2:
                nxt.append(vs[-1])
            vs = nxt
        return vs[0]

    def row_body(rr, carry):
        r = rk * ROWS_PER_TEC + rr
        fq0 = pb * NROW + r * H
        fq0_out = (3 - pb) * NROW + (H - 1 - r) * H
        grow = pb * H + r
        pltpu.sync_copy(idx_hbm.at[grow], idx_v)
        pltpu.sync_copy(mw_hbm.at[pl.ds(fq0, H)], mw_v)
        pltpu.sync_copy(sb_hbm.at[grow], sb_v)
        pltpu.async_copy(proj_hbm.at[idx_v.at[0]], rows_a, sem_a)
        for q in range(4):
            rows_q, sem_q = (rows_a, sem_a) if q % 2 == 0 else (rows_b, sem_b)
            if q < 3:
                rows_n, sem_n = (rows_b, sem_b) if q % 2 == 0 else (rows_a, sem_a)
                pltpu.async_copy(proj_hbm.at[idx_v.at[q + 1]], rows_n, sem_n)
            pltpu.make_async_copy(
                proj_hbm.at[idx_v.at[q]], rows_q, sem_q).wait()

            def sq_body(sq, c2, rows_q=rows_q, q=q):
                s = q * QS + sq
                base = sq * C
                for g in range(G):
                    mwv = mw_v[s, pl.ds(g * GC, GC)]
                    e = s * G + g
                    chv = sb_v[pl.ds((e >> 4) << 4, 16)]
                    sbv = chv.at[z16 + (e & 15)].get(mode="promise_in_bounds")
                    prods = [rows_q[base + g * GC + p, :] * bcast(mwv, p)
                             for p in range(P)]
                    prods.append(inb_gs[g] * sbv)
                    out_v[H - 1 - s, pl.ds(g * GC, GC)] = tree(prods)
                return c2

            lax.fori_loop(0, QS, sq_body, 0)
        pltpu.sync_copy(out_v, out_hbm.at[pl.ds(fq0_out, H)])
        return carry

    lax.fori_loop(0, ROWS_PER_TEC, row_body, 0)


def _sc_call(table, idx, mw, sb, inb):
    mesh = plsc.VectorSubcoreMesh(core_axis_name="c", subcore_axis_name="s")
    f = pl.kernel(
        _sc_body,
        out_type=jax.ShapeDtypeStruct((NPIX, C), jnp.float32),
        mesh=mesh,
        scratch_types=[
            pltpu.VMEM((4, QS * C), jnp.int32),
            pltpu.VMEM((H, C), jnp.float32),
            pltpu.VMEM((H * G,), jnp.float32),
            pltpu.VMEM((C,), jnp.float32),
            pltpu.VMEM((QS * C, GC), jnp.float32),
            pltpu.VMEM((QS * C, GC), jnp.float32),
            pltpu.VMEM((H, C), jnp.float32),
            pltpu.SemaphoreType.DMA,
            pltpu.SemaphoreType.DMA,
        ],
        compiler_params=pltpu.CompilerParams(use_tc_tiling_on_sc=False),
    )
    return f(table, idx, mw, sb, inb)


def kernel(input, dw_w, dw_b, ln_w, ln_b, off_w, off_b, mask_w, mask_b,
           in_w, in_b, out_w, out_b):
    inp = input[0]
    inpad = jnp.pad(inp, ((1, 1), (1, 1), (0, 0)))

    dwk = dw_w[:, 0]
    wcoef = jnp.stack([
        jnp.stack([dwk[:, int(2 * a + (pb // 2)), int(2 * b + (pb % 2))]
                   for a in (0, 1) for b in (0, 1)], axis=0)
        for pb in range(4)
    ], axis=0)

    wox = off_w[0::2].T
    woy = off_w[1::2].T
    wm = mask_w.T
    box = off_b[0::2]
    boy = off_b[1::2]
    bm = mask_b
    gid = jnp.arange(C) // GC
    bd = (gid[:, None] == gid[None, :]).astype(jnp.float32)
    bd4 = (gid[:, None] == jnp.arange(G)[None, :]).astype(jnp.float32)
    pch = jnp.arange(C) % P
    dxc = ((pch // 4) - 1).astype(jnp.float32)
    dyc = ((pch % 4) - 1).astype(jnp.float32)
    gofs = (jnp.arange(C) // GC * NROW).astype(jnp.int32)
    emat = (jnp.arange(G)[:, None] == gid[None, :]).astype(jnp.float32) * in_b[None, :]

    in_wt4 = in_w.T.reshape(C, G, GC).transpose(1, 0, 2)
    table = _proj_call(inp.reshape(NROW, C), in_wt4).reshape(G * NROW, GC)

    idx, mw, sb = _fields_call(inpad, wcoef, dw_b, ln_w, ln_b,
                               wox, woy, wm, box, boy, bm,
                               bd, bd4, dxc, dyc, gofs)

    dcn = _sc_call(table, idx.reshape(2 * HO, 4, QS * C), mw,
                   sb.reshape(2 * HO, H * G), in_b)

    out4 = _out_call(dcn, out_w.T, out_b)
    return out4.reshape(1, HO, HO, C)


# speedup vs baseline: 413.9389x; 1.0014x over previous
import functools
import math

import jax
import jax.numpy as jnp
from jax import lax
from jax.experimental import pallas as pl
from jax.experimental.pallas import tpu as pltpu
from jax.experimental.pallas import tpu_sc as plsc

C = 64
G = 4
GC = 16
P = 16
H = 112
HO = 224
NPIX = HO * HO
NROW = H * H

def _proj_body(inp_ref, wt_ref, out_ref):
    out_ref[:] = jnp.dot(inp_ref[:], wt_ref[0], preferred_element_type=jnp.float32)


def _proj_call(inp_flat, in_wt):
    return pl.pallas_call(
        _proj_body,
        grid=(G,),
        in_specs=[
            pl.BlockSpec((NROW, C), lambda g: (0, 0)),
            pl.BlockSpec((1, C, GC), lambda g: (g, 0, 0)),
        ],
        out_specs=pl.BlockSpec((NROW, GC), lambda g: (g, 0)),
        out_shape=jax.ShapeDtypeStruct((G * NROW, GC), jnp.float32),
    )(inp_flat, in_wt)


TR = 28
NRC = H // TR


def _axis_terms_tc(pos):
    i0 = jnp.floor(pos)
    fr = pos - i0
    ii = i0.astype(jnp.int32)
    odd = ii & 1
    ie = ii + odd
    wl = jnp.where(odd == 0, 1.0 - fr, fr)
    vl = (ie >= 2) & (ie <= HO)
    S = jnp.where((ii >= 0) & (ii <= 226), 1.0 - fr, 0.0) + \
        jnp.where((ii >= -1) & (ii <= 225), fr, 0.0)
    return ie, wl, vl, S


def _fields_body(inpad_ref, wcoef_ref, dwb_ref, lnw_ref, lnb_ref,
                 wox_ref, woy_ref, wm_ref, box_ref, boy_ref, bm_ref,
                 bd_ref, bd4_ref, dxc_ref, dyc_ref, gofs_ref,
                 idx_ref, mw_ref, sb_ref):
    pb = pl.program_id(0)
    rc = pl.program_id(1)
    pr = pb // 2
    pc = pb % 2
    r0 = rc * TR
    x = jnp.zeros((TR, H, C), jnp.float32)
    for ta in range(2):
        for tb in range(2):
            w = wcoef_ref[pb, ta * 2 + tb, :]
            rs = r0 + ta + pr
            cs = tb + pc
            sl = inpad_ref[pl.ds(rs, TR), pl.ds(cs, H), :]
            x = x + sl * w[None, None, :]
    x = x.reshape(TR * H, C) + dwb_ref[:]
    mu = jnp.mean(x, -1, keepdims=True)
    var = jnp.mean((x - mu) * (x - mu), -1, keepdims=True)
    x = (x - mu) * lax.rsqrt(var + 1e-6) * lnw_ref[:] + lnb_ref[:]
    x = 0.5 * x * (1.0 + lax.erf(x * (1.0 / math.sqrt(2.0))))
    offx = jnp.dot(x, wox_ref[:], preferred_element_type=jnp.float32) + box_ref[:]
    offy = jnp.dot(x, woy_ref[:], preferred_element_type=jnp.float32) + boy_ref[:]
    ml = jnp.dot(x, wm_ref[:], preferred_element_type=jnp.float32) + bm_ref[:]
    ml = ml - jnp.max(ml, -1, keepdims=True)
    e = jnp.exp(ml)
    m = e / jnp.dot(e, bd_ref[:], preferred_element_type=jnp.float32)

    blk = TR * H
    rI = lax.broadcasted_iota(jnp.int32, (TR, 1, 1), 0).astype(jnp.float32)
    sI = lax.broadcasted_iota(jnp.int32, (1, H, 1), 1).astype(jnp.float32)
    hof = ((HO + 1 - pr) - 2 * r0).astype(jnp.float32) - 2.0 * rI
    wof = (HO + 1 - pc).astype(jnp.float32) - 2.0 * sI
    py = (hof - dyc_ref[:][None, None, :]) - offy.reshape(TR, H, C)
    px = (wof - dxc_ref[:][None, None, :]) - offx.reshape(TR, H, C)
    ye, wy, vy, sy = _axis_terms_tc(py)
    xe, wx, vx, sx = _axis_terms_tc(px)
    valid = vy & vx
    w = jnp.where(valid, wy * wx, 0.0)
    rowi = ((ye - 2) >> 1) * H + ((xe - 2) >> 1)
    idxv = jnp.where(valid, rowi, 0).reshape(blk, C)
    idx_ref[:] = idxv + gofs_ref[:][None, :]
    mw_ref[:] = m * w.reshape(blk, C)
    sb_ref[:] = jnp.dot(m * (sy * sx).reshape(blk, C), bd4_ref[:],
                        preferred_element_type=jnp.float32)


def _fields_call(inpad, wcoef, dw_b, ln_w, ln_b, wox, woy, wm, box, boy, bm,
                 bd, bd4, dxc, dyc, gofs):
    blk = TR * H
    full = lambda shp: pl.BlockSpec(shp, lambda pb, rc: (0,) * len(shp))
    out_spec = pl.BlockSpec((blk, C), lambda pb, rc: (pb * NRC + rc, 0))
    return pl.pallas_call(
        _fields_body,
        grid=(4, NRC),
        in_specs=[
            full((H + 2, H + 2, C)),
            full((4, 4, C)),
            full((C,)), full((C,)), full((C,)),
            full((C, C)), full((C, C)), full((C, C)),
            full((C,)), full((C,)), full((C,)),
            full((C, C)), full((C, G)),
            full((C,)), full((C,)), full((C,)),
        ],
        out_specs=[
            out_spec, out_spec,
            pl.BlockSpec((blk, G), lambda pb, rc: (pb * NRC + rc, 0)),
        ],
        out_shape=[
            jax.ShapeDtypeStruct((NPIX, C), jnp.int32),
            jax.ShapeDtypeStruct((NPIX, C), jnp.float32),
            jax.ShapeDtypeStruct((NPIX, G), jnp.float32),
        ],
    )(inpad, wcoef, dw_b, ln_w, ln_b, wox, woy, wm, box, boy, bm,
      bd, bd4, dxc, dyc, gofs)


TRO = 28
NR4 = H // TRO


def _out_body(dcna_ref, dcnb_ref, wt_ref, b_ref, out_ref):
    oa = jnp.dot(dcna_ref[:], wt_ref[:], preferred_element_type=jnp.float32) + b_ref[:]
    ob = jnp.dot(dcnb_ref[:], wt_ref[:], preferred_element_type=jnp.float32) + b_ref[:]
    st = jnp.stack([oa.reshape(TRO, H, C), ob.reshape(TRO, H, C)], axis=2)
    out_ref[:] = st.reshape(TRO, 1, HO, C)


def _out_call(dcn, out_wt, out_b):
    blk = TRO * H
    return pl.pallas_call(
        _out_body,
        grid=(2, NR4),
        in_specs=[
            pl.BlockSpec((blk, C), lambda pr, rc: (2 * pr * NR4 + rc, 0)),
            pl.BlockSpec((blk, C), lambda pr, rc: ((2 * pr + 1) * NR4 + rc, 0)),
            pl.BlockSpec((C, C), lambda pr, rc: (0, 0)),
            pl.BlockSpec((C,), lambda pr, rc: (0,)),
        ],
        out_specs=pl.BlockSpec((TRO, 1, HO, C), lambda pr, rc: (rc, pr, 0, 0)),
        out_shape=jax.ShapeDtypeStruct((H, 2, HO, C), jnp.float32),
    )(dcn, dcn, out_wt, out_b)


ROWS_PER_TEC = 14
QS = 28


def _sc_body(proj_hbm, idx_hbm, mw_hbm, sb_hbm, inb_hbm, out_hbm,
             idx_v, mw_v, sb_v, inb_v, rows_a, rows_b, out_v, sem_a, sem_b):
    wid = lax.axis_index("s") * 2 + lax.axis_index("c")
    pb = wid // 8
    rk = wid % 8
    p16 = lax.iota(jnp.int32, 16)
    cps = [p16 * 0 + p for p in range(P)]
    z16 = p16 * 0
    pltpu.sync_copy(inb_hbm, inb_v)
    inb_gs = [inb_v[pl.ds(g * GC, GC)] for g in range(G)]

    def bcast(v, p):
        return v.at[cps[p]].get(mode="promise_in_bounds")

    def tree(vs):
        while len(vs) > 1:
            nxt = [vs[i] + vs[i + 1] for i in range(0, len(vs) - 1, 2)]
            if len(vs) % 2:
                nxt.append(vs[-1])
            vs = nxt
        return vs[0]

    def row_body(rr, carry):
        r = rk * ROWS_PER_TEC + rr
        fq0 = pb * NROW + r * H
        fq0_out = (3 - pb) * NROW + (H - 1 - r) * H
        grow = pb * H + r
        pltpu.sync_copy(idx_hbm.at[grow], idx_v)
        pltpu.sync_copy(mw_hbm.at[pl.ds(fq0, H)], mw_v)
        pltpu.sync_copy(sb_hbm.at[grow], sb_v)
        pltpu.async_copy(proj_hbm.at[idx_v.at[0]], rows_a, sem_a)
        for q in range(4):
            rows_q, sem_q = (rows_a, sem_a) if q % 2 == 0 else (rows_b, sem_b)
            if q < 3:
                rows_n, sem_n = (rows_b, sem_b) if q % 2 == 0 else (rows_a, sem_a)
                pltpu.async_copy(proj_hbm.at[idx_v.at[q + 1]], rows_n, sem_n)
            pltpu.make_async_copy(
                proj_hbm.at[idx_v.at[q]], rows_q, sem_q).wait()

            def sq_body(sq, c2, rows_q=rows_q, q=q):
                s = q * QS + sq
                base = sq * C
                for g in range(G):
                    mwv = mw_v[s, pl.ds(g * GC, GC)]
                    e = s * G + g
                    chv = sb_v[pl.ds((e >> 4) << 4, 16)]
                    sbv = chv.at[z16 + (e & 15)].get(mode="promise_in_bounds")
                    prods = [rows_q[base + g * GC + p, :] * bcast(mwv, p)
                             for p in range(P)]
                    prods.append(inb_gs[g] * sbv)
                    out_v[H - 1 - s, pl.ds(g * GC, GC)] = tree(prods)
                return c2

            lax.fori_loop(0, QS, sq_body, 0)
        pltpu.sync_copy(out_v, out_hbm.at[pl.ds(fq0_out, H)])
        return carry

    lax.fori_loop(0, ROWS_PER_TEC, row_body, 0)


def _sc_call(table, idx, mw, sb, inb):
    mesh = plsc.VectorSubcoreMesh(core_axis_name="c", subcore_axis_name="s")
    f = pl.kernel(
        _sc_body,
        out_type=jax.ShapeDtypeStruct((NPIX, C), jnp.float32),
        mesh=mesh,
        scratch_types=[
            pltpu.VMEM((4, QS * C), jnp.int32),
            pltpu.VMEM((H, C), jnp.float32),
            pltpu.VMEM((H * G,), jnp.float32),
            pltpu.VMEM((C,), jnp.float32),
            pltpu.VMEM((QS * C, GC), jnp.float32),
            pltpu.VMEM((QS * C, GC), jnp.float32),
            pltpu.VMEM((H, C), jnp.float32),
            pltpu.SemaphoreType.DMA,
            pltpu.SemaphoreType.DMA,
        ],
        compiler_params=pltpu.CompilerParams(use_tc_tiling_on_sc=False),
    )
    return f(table, idx, mw, sb, inb)


def kernel(input, dw_w, dw_b, ln_w, ln_b, off_w, off_b, mask_w, mask_b,
           in_w, in_b, out_w, out_b):
    inp = input[0]
    inpad = jnp.pad(inp, ((1, 1), (1, 1), (0, 0)))

    dwk = dw_w[:, 0]
    wcoef = jnp.stack([
        jnp.stack([dwk[:, int(2 * a + (pb // 2)), int(2 * b + (pb % 2))]
                   for a in (0, 1) for b in (0, 1)], axis=0)
        for pb in range(4)
    ], axis=0)

    wox = off_w[0::2].T
    woy = off_w[1::2].T
    wm = mask_w.T
    box = off_b[0::2]
    boy = off_b[1::2]
    bm = mask_b
    gid = jnp.arange(C) // GC
    bd = (gid[:, None] == gid[None, :]).astype(jnp.float32)
    bd4 = (gid[:, None] == jnp.arange(G)[None, :]).astype(jnp.float32)
    pch = jnp.arange(C) % P
    dxc = ((pch // 4) - 1).astype(jnp.float32)
    dyc = ((pch % 4) - 1).astype(jnp.float32)
    gofs = (jnp.arange(C) // GC * NROW).astype(jnp.int32)
    emat = (jnp.arange(G)[:, None] == gid[None, :]).astype(jnp.float32) * in_b[None, :]

    in_wt4 = in_w.T.reshape(C, G, GC).transpose(1, 0, 2)
    table = _proj_call(inp.reshape(NROW, C), in_wt4)

    idx, mw, sb = _fields_call(inpad, wcoef, dw_b, ln_w, ln_b,
                               wox, woy, wm, box, boy, bm,
                               bd, bd4, dxc, dyc, gofs)

    dcn = _sc_call(table, idx.reshape(2 * HO, 4, QS * C), mw,
                   sb.reshape(2 * HO, H * G), in_b)

    out4 = _out_call(dcn, out_w.T, out_b)
    return out4.reshape(1, HO, HO, C)


# speedup vs baseline: 414.1503x; 1.0005x over previous
import functools
import math

import jax
import jax.numpy as jnp
from jax import lax
from jax.experimental import pallas as pl
from jax.experimental.pallas import tpu as pltpu
from jax.experimental.pallas import tpu_sc as plsc

C = 64
G = 4
GC = 16
P = 16
H = 112
HO = 224
NPIX = HO * HO
NROW = H * H

def _proj_body(inp_ref, wt_ref, out_ref):
    out_ref[:] = jnp.dot(inp_ref[:], wt_ref[0], preferred_element_type=jnp.float32)


def _proj_call(inp_flat, in_wt):
    return pl.pallas_call(
        _proj_body,
        grid=(G,),
        in_specs=[
            pl.BlockSpec((NROW, C), lambda g: (0, 0)),
            pl.BlockSpec((1, C, GC), lambda g: (g, 0, 0)),
        ],
        out_specs=pl.BlockSpec((NROW, GC), lambda g: (g, 0)),
        out_shape=jax.ShapeDtypeStruct((G * NROW, GC), jnp.float32),
    )(inp_flat, in_wt)


TR = 28
NRC = H // TR


def _axis_terms_tc(pos):
    i0 = jnp.floor(pos)
    fr = pos - i0
    ii = i0.astype(jnp.int32)
    odd = ii & 1
    ie = ii + odd
    wl = jnp.where(odd == 0, 1.0 - fr, fr)
    vl = (ie >= 2) & (ie <= HO)
    S = jnp.where((ii >= 0) & (ii <= 226), 1.0 - fr, 0.0) + \
        jnp.where((ii >= -1) & (ii <= 225), fr, 0.0)
    return ie, wl, vl, S


def _fields_body(inpad_ref, wcoef_ref, dwb_ref, lnw_ref, lnb_ref,
                 wox_ref, woy_ref, wm_ref, box_ref, boy_ref, bm_ref,
                 bd_ref, bd4_ref, dxc_ref, dyc_ref, gofs_ref,
                 idx_ref, mw_ref, sb_ref):
    pb = pl.program_id(0)
    rc = pl.program_id(1)
    pr = pb // 2
    pc = pb % 2
    r0 = rc * TR
    x = jnp.zeros((TR, H, C), jnp.float32)
    for ta in range(2):
        for tb in range(2):
            w = wcoef_ref[pb, ta * 2 + tb, :]
            rs = r0 + ta + pr
            cs = tb + pc
            sl = inpad_ref[pl.ds(rs, TR), pl.ds(cs, H), :]
            x = x + sl * w[None, None, :]
    x = x.reshape(TR * H, C) + dwb_ref[:]
    mu = jnp.mean(x, -1, keepdims=True)
    var = jnp.mean((x - mu) * (x - mu), -1, keepdims=True)
    x = (x - mu) * lax.rsqrt(var + 1e-6) * lnw_ref[:] + lnb_ref[:]
    x = 0.5 * x * (1.0 + lax.erf(x * (1.0 / math.sqrt(2.0))))
    offx = jnp.dot(x, wox_ref[:], preferred_element_type=jnp.float32) + box_ref[:]
    offy = jnp.dot(x, woy_ref[:], preferred_element_type=jnp.float32) + boy_ref[:]
    ml = jnp.dot(x, wm_ref[:], preferred_element_type=jnp.float32) + bm_ref[:]
    ml = ml - jnp.max(ml, -1, keepdims=True)
    e = jnp.exp(ml)
    m = e / jnp.dot(e, bd_ref[:], preferred_element_type=jnp.float32)

    blk = TR * H
    rI = lax.broadcasted_iota(jnp.int32, (TR, 1, 1), 0).astype(jnp.float32)
    sI = lax.broadcasted_iota(jnp.int32, (1, H, 1), 1).astype(jnp.float32)
    hof = ((HO + 1 - pr) - 2 * r0).astype(jnp.float32) - 2.0 * rI
    wof = (HO + 1 - pc).astype(jnp.float32) - 2.0 * sI
    py = (hof - dyc_ref[:][None, None, :]) - offy.reshape(TR, H, C)
    px = (wof - dxc_ref[:][None, None, :]) - offx.reshape(TR, H, C)
    ye, wy, vy, sy = _axis_terms_tc(py)
    xe, wx, vx, sx = _axis_terms_tc(px)
    valid = vy & vx
    w = jnp.where(valid, wy * wx, 0.0)
    rowi = ((ye - 2) >> 1) * H + ((xe - 2) >> 1)
    idxv = jnp.where(valid, rowi, 0).reshape(blk, C)
    idx_ref[:] = idxv + gofs_ref[:][None, :]
    mw_ref[:] = m * w.reshape(blk, C)
    sb_ref[:] = jnp.dot(m * (sy * sx).reshape(blk, C), bd4_ref[:],
                        preferred_element_type=jnp.float32)


def _fields_call(inpad, wcoef, dw_b, ln_w, ln_b, wox, woy, wm, box, boy, bm,
                 bd, bd4, dxc, dyc, gofs):
    blk = TR * H
    full = lambda shp: pl.BlockSpec(shp, lambda pb, rc: (0,) * len(shp))
    out_spec = pl.BlockSpec((blk, C), lambda pb, rc: (pb * NRC + rc, 0))
    return pl.pallas_call(
        _fields_body,
        grid=(4, NRC),
        in_specs=[
            full((H + 2, H + 2, C)),
            full((4, 4, C)),
            full((C,)), full((C,)), full((C,)),
            full((C, C)), full((C, C)), full((C, C)),
            full((C,)), full((C,)), full((C,)),
            full((C, C)), full((C, G)),
            full((C,)), full((C,)), full((C,)),
        ],
        out_specs=[
            out_spec, out_spec,
            pl.BlockSpec((blk, G), lambda pb, rc: (pb * NRC + rc, 0)),
        ],
        out_shape=[
            jax.ShapeDtypeStruct((NPIX, C), jnp.int32),
            jax.ShapeDtypeStruct((NPIX, C), jnp.float32),
            jax.ShapeDtypeStruct((NPIX, G), jnp.float32),
        ],
    )(inpad, wcoef, dw_b, ln_w, ln_b, wox, woy, wm, box, boy, bm,
      bd, bd4, dxc, dyc, gofs)


TRO = 28
NR4 = H // TRO


def _out_body(dcna_ref, dcnb_ref, wt_ref, b_ref, out_ref):
    oa = jnp.dot(dcna_ref[:], wt_ref[:], preferred_element_type=jnp.float32) + b_ref[:]
    ob = jnp.dot(dcnb_ref[:], wt_ref[:], preferred_element_type=jnp.float32) + b_ref[:]
    st = jnp.stack([oa.reshape(TRO, H, C), ob.reshape(TRO, H, C)], axis=2)
    out_ref[:] = st.reshape(TRO, 1, HO, C)


def _out_call(dcn, out_wt, out_b):
    blk = TRO * H
    return pl.pallas_call(
        _out_body,
        grid=(2, NR4),
        in_specs=[
            pl.BlockSpec((blk, C), lambda pr, rc: (2 * pr * NR4 + rc, 0)),
            pl.BlockSpec((blk, C), lambda pr, rc: ((2 * pr + 1) * NR4 + rc, 0)),
            pl.BlockSpec((C, C), lambda pr, rc: (0, 0)),
            pl.BlockSpec((C,), lambda pr, rc: (0,)),
        ],
        out_specs=pl.BlockSpec((TRO, 1, HO, C), lambda pr, rc: (rc, pr, 0, 0)),
        out_shape=jax.ShapeDtypeStruct((H, 2, HO, C), jnp.float32),
    )(dcn, dcn, out_wt, out_b)


ROWS_PER_TEC = 14
QS = 28


def _sc_body(proj_hbm, idx_hbm, mw_hbm, sb_hbm, inb_hbm, out_hbm,
             idx_v, mw_v, sb_v, inb_v, rows_a, rows_b, out_v, sem_a, sem_b):
    wid = lax.axis_index("s") * 2 + lax.axis_index("c")
    pb = wid // 8
    rk = wid % 8
    p16 = lax.iota(jnp.int32, 16)
    cps = [p16 * 0 + p for p in range(P)]
    z16 = p16 * 0
    pltpu.sync_copy(inb_hbm, inb_v)
    inb_gs = [inb_v[pl.ds(g * GC, GC)] for g in range(G)]

    def bcast(v, p):
        return v.at[cps[p]].get(mode="promise_in_bounds")

    def tree(vs):
        while len(vs) > 1:
            nxt = [vs[i] + vs[i + 1] for i in range(0, len(vs) - 1, 2)]
            if len(vs) % 2:
                nxt.append(vs[-1])
            vs = nxt
        return vs[0]

    def row_body(rr, carry):
        r = rk * ROWS_PER_TEC + rr
        fq0 = pb * NROW + r * H
        fq0_out = (3 - pb) * NROW + (H - 1 - r) * H
        grow = pb * H + r
        pltpu.sync_copy(idx_hbm.at[grow], idx_v)
        pltpu.sync_copy(mw_hbm.at[pl.ds(fq0, H)], mw_v)
        pltpu.sync_copy(sb_hbm.at[grow], sb_v)
        pltpu.async_copy(proj_hbm.at[idx_v.at[0]], rows_a, sem_a)
        for q in range(4):
            rows_q, sem_q = (rows_a, sem_a) if q % 2 == 0 else (rows_b, sem_b)
            if q < 3:
                rows_n, sem_n = (rows_b, sem_b) if q % 2 == 0 else (rows_a, sem_a)
                pltpu.async_copy(proj_hbm.at[idx_v.at[q + 1]], rows_n, sem_n)
            pltpu.make_async_copy(
                proj_hbm.at[idx_v.at[q]], rows_q, sem_q).wait()

            def sq_body(sq, c2, rows_q=rows_q, q=q):
                s = q * QS + sq
                base = sq * C
                for g in range(G):
                    mwv = mw_v[s, pl.ds(g * GC, GC)]
                    e = s * G + g
                    chv = sb_v[pl.ds((e >> 4) << 4, 16)]
                    sbv = chv.at[z16 + (e & 15)].get(mode="promise_in_bounds")
                    prods = [rows_q[base + g * GC + p, :] * bcast(mwv, p)
                             for p in range(P)]
                    prods.append(inb_gs[g] * sbv)
                    out_v[H - 1 - s, pl.ds(g * GC, GC)] = tree(prods)
                return c2

            lax.fori_loop(0, QS, sq_body, 0, unroll=2)
        pltpu.sync_copy(out_v, out_hbm.at[pl.ds(fq0_out, H)])
        return carry

    lax.fori_loop(0, ROWS_PER_TEC, row_body, 0)


def _sc_call(table, idx, mw, sb, inb):
    mesh = plsc.VectorSubcoreMesh(core_axis_name="c", subcore_axis_name="s")
    f = pl.kernel(
        _sc_body,
        out_type=jax.ShapeDtypeStruct((NPIX, C), jnp.float32),
        mesh=mesh,
        scratch_types=[
            pltpu.VMEM((4, QS * C), jnp.int32),
            pltpu.VMEM((H, C), jnp.float32),
            pltpu.VMEM((H * G,), jnp.float32),
            pltpu.VMEM((C,), jnp.float32),
            pltpu.VMEM((QS * C, GC), jnp.float32),
            pltpu.VMEM((QS * C, GC), jnp.float32),
            pltpu.VMEM((H, C), jnp.float32),
            pltpu.SemaphoreType.DMA,
            pltpu.SemaphoreType.DMA,
        ],
        compiler_params=pltpu.CompilerParams(use_tc_tiling_on_sc=False),
    )
    return f(table, idx, mw, sb, inb)


def kernel(input, dw_w, dw_b, ln_w, ln_b, off_w, off_b, mask_w, mask_b,
           in_w, in_b, out_w, out_b):
    inp = input[0]
    inpad = jnp.pad(inp, ((1, 1), (1, 1), (0, 0)))

    dwk = dw_w[:, 0]
    wcoef = jnp.stack([
        jnp.stack([dwk[:, int(2 * a + (pb // 2)), int(2 * b + (pb % 2))]
                   for a in (0, 1) for b in (0, 1)], axis=0)
        for pb in range(4)
    ], axis=0)

    wox = off_w[0::2].T
    woy = off_w[1::2].T
    wm = mask_w.T
    box = off_b[0::2]
    boy = off_b[1::2]
    bm = mask_b
    gid = jnp.arange(C) // GC
    bd = (gid[:, None] == gid[None, :]).astype(jnp.float32)
    bd4 = (gid[:, None] == jnp.arange(G)[None, :]).astype(jnp.float32)
    pch = jnp.arange(C) % P
    dxc = ((pch // 4) - 1).astype(jnp.float32)
    dyc = ((pch % 4) - 1).astype(jnp.float32)
    gofs = (jnp.arange(C) // GC * NROW).astype(jnp.int32)
    emat = (jnp.arange(G)[:, None] == gid[None, :]).astype(jnp.float32) * in_b[None, :]

    in_wt4 = in_w.T.reshape(C, G, GC).transpose(1, 0, 2)
    table = _proj_call(inp.reshape(NROW, C), in_wt4)

    idx, mw, sb = _fields_call(inpad, wcoef, dw_b, ln_w, ln_b,
                               wox, woy, wm, box, boy, bm,
                               bd, bd4, dxc, dyc, gofs)

    dcn = _sc_call(table, idx.reshape(2 * HO, 4, QS * C), mw,
                   sb.reshape(2 * HO, H * G), in_b)

    out4 = _out_call(dcn, out_w.T, out_b)
    return out4.reshape(1, HO, HO, C)


# speedup vs baseline: 433.5899x; 1.0469x over previous
import functools
import math

import jax
import jax.numpy as jnp
from jax import lax
from jax.experimental import pallas as pl
from jax.experimental.pallas import tpu as pltpu
from jax.experimental.pallas import tpu_sc as plsc

C = 64
G = 4
GC = 16
P = 16
H = 112
HO = 224
NPIX = HO * HO
NROW = H * H

def _proj_body(inp_ref, wt_ref, out_ref):
    out_ref[:] = jnp.dot(inp_ref[:], wt_ref[0], preferred_element_type=jnp.float32)


def _proj_call(inp_flat, in_wt):
    return pl.pallas_call(
        _proj_body,
        grid=(G,),
        in_specs=[
            pl.BlockSpec((NROW, C), lambda g: (0, 0)),
            pl.BlockSpec((1, C, GC), lambda g: (g, 0, 0)),
        ],
        out_specs=pl.BlockSpec((NROW, GC), lambda g: (g, 0)),
        out_shape=jax.ShapeDtypeStruct((G * NROW, GC), jnp.float32),
    )(inp_flat, in_wt)


TR = 28
NRC = H // TR


def _axis_terms_tc(pos):
    i0 = jnp.floor(pos)
    fr = pos - i0
    ii = i0.astype(jnp.int32)
    odd = ii & 1
    ie = ii + odd
    wl = jnp.where(odd == 0, 1.0 - fr, fr)
    vl = (ie >= 2) & (ie <= HO)
    S = jnp.where((ii >= 0) & (ii <= 226), 1.0 - fr, 0.0) + \
        jnp.where((ii >= -1) & (ii <= 225), fr, 0.0)
    return ie, wl, vl, S


def _fields_body(inpad_ref, wcoef_ref, dwb_ref, lnw_ref, lnb_ref,
                 wox_ref, woy_ref, wm_ref, box_ref, boy_ref, bm_ref,
                 bd_ref, bd4_ref, dxc_ref, dyc_ref, gofs_ref,
                 idx_ref, mw_ref, sb_ref):
    pb = pl.program_id(0)
    rc = pl.program_id(1)
    pr = pb // 2
    pc = pb % 2
    r0 = rc * TR
    x = jnp.zeros((TR, H, C), jnp.float32)
    for ta in range(2):
        for tb in range(2):
            w = wcoef_ref[pb, ta * 2 + tb, :]
            rs = r0 + ta + pr
            cs = tb + pc
            sl = inpad_ref[pl.ds(rs, TR), pl.ds(cs, H), :]
            x = x + sl * w[None, None, :]
    x = x.reshape(TR * H, C) + dwb_ref[:]
    mu = jnp.mean(x, -1, keepdims=True)
    var = jnp.mean((x - mu) * (x - mu), -1, keepdims=True)
    x = (x - mu) * lax.rsqrt(var + 1e-6) * lnw_ref[:] + lnb_ref[:]
    x = 0.5 * x * (1.0 + lax.erf(x * (1.0 / math.sqrt(2.0))))
    offx = jnp.dot(x, wox_ref[:], preferred_element_type=jnp.float32) + box_ref[:]
    offy = jnp.dot(x, woy_ref[:], preferred_element_type=jnp.float32) + boy_ref[:]
    ml = jnp.dot(x, wm_ref[:], preferred_element_type=jnp.float32) + bm_ref[:]
    ml = ml - jnp.max(ml, -1, keepdims=True)
    e = jnp.exp(ml)
    m = e / jnp.dot(e, bd_ref[:], preferred_element_type=jnp.float32)

    blk = TR * H
    rI = lax.broadcasted_iota(jnp.int32, (TR, 1, 1), 0).astype(jnp.float32)
    sI = lax.broadcasted_iota(jnp.int32, (1, H, 1), 1).astype(jnp.float32)
    hof = ((HO + 1 - pr) - 2 * r0).astype(jnp.float32) - 2.0 * rI
    wof = (HO + 1 - pc).astype(jnp.float32) - 2.0 * sI
    py = (hof - dyc_ref[:][None, None, :]) - offy.reshape(TR, H, C)
    px = (wof - dxc_ref[:][None, None, :]) - offx.reshape(TR, H, C)
    ye, wy, vy, sy = _axis_terms_tc(py)
    xe, wx, vx, sx = _axis_terms_tc(px)
    valid = vy & vx
    w = jnp.where(valid, wy * wx, 0.0)
    rowi = ((ye - 2) >> 1) * H + ((xe - 2) >> 1)
    idxv = jnp.where(valid, rowi, 0).reshape(blk, C)
    idx_ref[:] = idxv + gofs_ref[:][None, :]
    mw_ref[:] = m * w.reshape(blk, C)
    sb_ref[:] = jnp.dot(m * (sy * sx).reshape(blk, C), bd4_ref[:],
                        preferred_element_type=jnp.float32)


def _fields_call(inpad, wcoef, dw_b, ln_w, ln_b, wox, woy, wm, box, boy, bm,
                 bd, bd4, dxc, dyc, gofs):
    blk = TR * H
    full = lambda shp: pl.BlockSpec(shp, lambda pb, rc: (0,) * len(shp))
    out_spec = pl.BlockSpec((blk, C), lambda pb, rc: (pb * NRC + rc, 0))
    return pl.pallas_call(
        _fields_body,
        grid=(4, NRC),
        in_specs=[
            full((H + 2, H + 2, C)),
            full((4, 4, C)),
            full((C,)), full((C,)), full((C,)),
            full((C, C)), full((C, C)), full((C, C)),
            full((C,)), full((C,)), full((C,)),
            full((C, C)), full((C, G)),
            full((C,)), full((C,)), full((C,)),
        ],
        out_specs=[
            out_spec, out_spec,
            pl.BlockSpec((blk, G), lambda pb, rc: (pb * NRC + rc, 0)),
        ],
        out_shape=[
            jax.ShapeDtypeStruct((NPIX, C), jnp.int32),
            jax.ShapeDtypeStruct((NPIX, C), jnp.float32),
            jax.ShapeDtypeStruct((NPIX, G), jnp.float32),
        ],
    )(inpad, wcoef, dw_b, ln_w, ln_b, wox, woy, wm, box, boy, bm,
      bd, bd4, dxc, dyc, gofs)


TRO = 28
NR4 = H // TRO


def _out_body(dcna_ref, dcnb_ref, wt_ref, b_ref, out_ref):
    oa = jnp.dot(dcna_ref[:], wt_ref[:], preferred_element_type=jnp.float32) + b_ref[:]
    ob = jnp.dot(dcnb_ref[:], wt_ref[:], preferred_element_type=jnp.float32) + b_ref[:]
    st = jnp.stack([oa.reshape(TRO, H, C), ob.reshape(TRO, H, C)], axis=2)
    out_ref[:] = st.reshape(TRO, 1, HO, C)


def _out_call(dcn, out_wt, out_b):
    blk = TRO * H
    return pl.pallas_call(
        _out_body,
        grid=(2, NR4),
        in_specs=[
            pl.BlockSpec((blk, C), lambda pr, rc: (2 * pr * NR4 + rc, 0)),
            pl.BlockSpec((blk, C), lambda pr, rc: ((2 * pr + 1) * NR4 + rc, 0)),
            pl.BlockSpec((C, C), lambda pr, rc: (0, 0)),
            pl.BlockSpec((C,), lambda pr, rc: (0,)),
        ],
        out_specs=pl.BlockSpec((TRO, 1, HO, C), lambda pr, rc: (rc, pr, 0, 0)),
        out_shape=jax.ShapeDtypeStruct((H, 2, HO, C), jnp.float32),
    )(dcn, dcn, out_wt, out_b)


ROWS_PER_TEC = 14
QS = 28


def _sc_body(proj_hbm, idx_hbm, mw_hbm, sb_hbm, inb_hbm, out_hbm,
             idx_v, mw_v, sb_v, idx_w, mw_w, sb_w, inb_v, rows_a, rows_b,
             out_v, sem_a, sem_b, sem_s):
    wid = lax.axis_index("s") * 2 + lax.axis_index("c")
    pb = wid // 8
    rk = wid % 8
    p16 = lax.iota(jnp.int32, 16)
    cps = [p16 * 0 + p for p in range(P)]
    z16 = p16 * 0
    pltpu.sync_copy(inb_hbm, inb_v)
    inb_gs = [inb_v[pl.ds(g * GC, GC)] for g in range(G)]

    def bcast(v, p):
        return v.at[cps[p]].get(mode="promise_in_bounds")

    def tree(vs):
        while len(vs) > 1:
            nxt = [vs[i] + vs[i + 1] for i in range(0, len(vs) - 1, 2)]
            if len(vs) % 2:
                nxt.append(vs[-1])
            vs = nxt
        return vs[0]

    def stage(r, bufs):
        grow = pb * H + r
        fq0 = pb * NROW + r * H
        pltpu.async_copy(idx_hbm.at[grow], bufs[0], sem_s)
        pltpu.async_copy(mw_hbm.at[pl.ds(fq0, H)], bufs[1], sem_s)
        pltpu.async_copy(sb_hbm.at[grow], bufs[2], sem_s)

    def wait_stage(r, bufs):
        grow = pb * H + r
        fq0 = pb * NROW + r * H
        pltpu.make_async_copy(idx_hbm.at[grow], bufs[0], sem_s).wait()
        pltpu.make_async_copy(mw_hbm.at[pl.ds(fq0, H)], bufs[1], sem_s).wait()
        pltpu.make_async_copy(sb_hbm.at[grow], bufs[2], sem_s).wait()

    def process(r, bufs):
        idx_c, mw_c, sb_c = bufs
        fq0 = pb * NROW + r * H
        fq0_out = (3 - pb) * NROW + (H - 1 - r) * H
        pltpu.async_copy(proj_hbm.at[idx_c.at[0]], rows_a, sem_a)
        for q in range(4):
            rows_q, sem_q = (rows_a, sem_a) if q % 2 == 0 else (rows_b, sem_b)
            if q < 3:
                rows_n, sem_n = (rows_b, sem_b) if q % 2 == 0 else (rows_a, sem_a)
                pltpu.async_copy(proj_hbm.at[idx_c.at[q + 1]], rows_n, sem_n)
            pltpu.make_async_copy(
                proj_hbm.at[idx_c.at[q]], rows_q, sem_q).wait()

            def sq_body(sq, c2, rows_q=rows_q, q=q):
                s = q * QS + sq
                base = sq * C
                for g in range(G):
                    mwv = mw_c[s, pl.ds(g * GC, GC)]
                    e = s * G + g
                    chv = sb_c[pl.ds((e >> 4) << 4, 16)]
                    sbv = chv.at[z16 + (e & 15)].get(mode="promise_in_bounds")
                    prods = [rows_q[base + g * GC + p, :] * bcast(mwv, p)
                             for p in range(P)]
                    prods.append(inb_gs[g] * sbv)
                    out_v[H - 1 - s, pl.ds(g * GC, GC)] = tree(prods)
                return c2

            lax.fori_loop(0, QS, sq_body, 0, unroll=2)
        pltpu.sync_copy(out_v, out_hbm.at[pl.ds(fq0_out, H)])

    bufs_a = (idx_v, mw_v, sb_v)
    bufs_b = (idx_w, mw_w, sb_w)
    r0 = rk * ROWS_PER_TEC
    stage(r0, bufs_a)

    def pair_body(k, carry):
        r = r0 + 2 * k
        wait_stage(r, bufs_a)
        stage(r + 1, bufs_b)
        process(r, bufs_a)
        wait_stage(r + 1, bufs_b)

        @pl.when(k < ROWS_PER_TEC // 2 - 1)
        def _():
            stage(r + 2, bufs_a)

        process(r + 1, bufs_b)
        return carry

    lax.fori_loop(0, ROWS_PER_TEC // 2, pair_body, 0)


def _sc_call(table, idx, mw, sb, inb):
    mesh = plsc.VectorSubcoreMesh(core_axis_name="c", subcore_axis_name="s")
    f = pl.kernel(
        _sc_body,
        out_type=jax.ShapeDtypeStruct((NPIX, C), jnp.float32),
        mesh=mesh,
        scratch_types=[
            pltpu.VMEM((4, QS * C), jnp.int32),
            pltpu.VMEM((H, C), jnp.float32),
            pltpu.VMEM((H * G,), jnp.float32),
            pltpu.VMEM((4, QS * C), jnp.int32),
            pltpu.VMEM((H, C), jnp.float32),
            pltpu.VMEM((H * G,), jnp.float32),
            pltpu.VMEM((C,), jnp.float32),
            pltpu.VMEM((QS * C, GC), jnp.float32),
            pltpu.VMEM((QS * C, GC), jnp.float32),
            pltpu.VMEM((H, C), jnp.float32),
            pltpu.SemaphoreType.DMA,
            pltpu.SemaphoreType.DMA,
            pltpu.SemaphoreType.DMA,
        ],
        compiler_params=pltpu.CompilerParams(use_tc_tiling_on_sc=False),
    )
    return f(table, idx, mw, sb, inb)


def kernel(input, dw_w, dw_b, ln_w, ln_b, off_w, off_b, mask_w, mask_b,
           in_w, in_b, out_w, out_b):
    inp = input[0]
    inpad = jnp.pad(inp, ((1, 1), (1, 1), (0, 0)))

    dwk = dw_w[:, 0]
    wcoef = jnp.stack([
        jnp.stack([dwk[:, int(2 * a + (pb // 2)), int(2 * b + (pb % 2))]
                   for a in (0, 1) for b in (0, 1)], axis=0)
        for pb in range(4)
    ], axis=0)

    wox = off_w[0::2].T
    woy = off_w[1::2].T
    wm = mask_w.T
    box = off_b[0::2]
    boy = off_b[1::2]
    bm = mask_b
    gid = jnp.arange(C) // GC
    bd = (gid[:, None] == gid[None, :]).astype(jnp.float32)
    bd4 = (gid[:, None] == jnp.arange(G)[None, :]).astype(jnp.float32)
    pch = jnp.arange(C) % P
    dxc = ((pch // 4) - 1).astype(jnp.float32)
    dyc = ((pch % 4) - 1).astype(jnp.float32)
    gofs = (jnp.arange(C) // GC * NROW).astype(jnp.int32)
    emat = (jnp.arange(G)[:, None] == gid[None, :]).astype(jnp.float32) * in_b[None, :]

    in_wt4 = in_w.T.reshape(C, G, GC).transpose(1, 0, 2)
    table = _proj_call(inp.reshape(NROW, C), in_wt4)

    idx, mw, sb = _fields_call(inpad, wcoef, dw_b, ln_w, ln_b,
                               wox, woy, wm, box, boy, bm,
                               bd, bd4, dxc, dyc, gofs)

    dcn = _sc_call(table, idx.reshape(2 * HO, 4, QS * C), mw,
                   sb.reshape(2 * HO, H * G), in_b)

    out4 = _out_call(dcn, out_w.T, out_b)
    return out4.reshape(1, HO, HO, C)
